# Initial kernel scaffold; baseline (speedup 1.0000x reference)
import jax
import jax.numpy as jnp
from jax.experimental import pallas as pl


def kernel(x, transformation):
    raise NotImplementedError("write your pallas kernel here")


# speedup vs baseline: 1.3337x; 1.3337x over previous
import functools

import jax
import jax.numpy as jnp
from jax import lax
from jax.experimental import pallas as pl
from jax.experimental.pallas import tpu as pltpu
from jax.experimental.pallas import tpu_sc as plsc

B, H, W, C = 8, 224, 224, 96
OUT_H, OUT_W = 224, 224
HW = H * W
NC, NS, L = 2, 16, 16
NW = NC * NS
PIX_PER_TILE = B * OUT_H * OUT_W // NW
ROWS_PER_TILE = PIX_PER_TILE // OUT_W
TILES_PER_BATCH = OUT_H // ROWS_PER_TILE
CHUNK = 112
NVEC = CHUNK // L
CHUNKS_PER_ROW = OUT_W // CHUNK
N_CHUNKS = PIX_PER_TILE // CHUNK


_SPLAT_DNUMS = lax.GatherDimensionNumbers(
    offset_dims=(), collapsed_slice_dims=(0,), start_index_map=(0,))


def _lane_splat(vec, l):
    idx = jnp.full((L, 1), l, jnp.int32)
    return lax.gather(vec, idx, _SPLAT_DNUMS, slice_sizes=(1,),
                      mode=lax.GatherScatterMode.PROMISE_IN_BOUNDS)


def _lane_splat_dyn(vec, lane):
    idx = (lax.iota(jnp.int32, L) * 0 + lane).reshape(L, 1)
    return lax.gather(vec, idx, _SPLAT_DNUMS, slice_sizes=(1,),
                      mode=lax.GatherScatterMode.PROMISE_IN_BOUNDS)


def _body(x_hbm, th_hbm, lin_hbm, out_hbm,
          ia_r, ib_r, ic_r, id_r, w_r,
          ga_r, gb_r, gc_r, gd_r, o_r, th_v, lin_v, sem):
    cid = lax.axis_index("c")
    sid = lax.axis_index("s")
    wid = sid * NC + cid
    b = wid // TILES_PER_BATCH
    q = wid % TILES_PER_BATCH

    pltpu.sync_copy(th_hbm.at[b], th_v)
    pltpu.sync_copy(lin_hbm, lin_v)
    thvec = th_v[...]
    t00 = _lane_splat(thvec, 0)
    t01 = _lane_splat(thvec, 1)
    t02 = _lane_splat(thvec, 2)
    t10 = _lane_splat(thvec, 3)
    t11 = _lane_splat(thvec, 4)
    t12 = _lane_splat(thvec, 5)

    base_idx = b * HW

    def row_body(r, carry):
        row = q * ROWS_PER_TILE + r
        rbase = (row // L) * L
        yc = _lane_splat_dyn(lin_v[pl.ds(rbase, L)], row - rbase)
        for half in range(CHUNKS_PER_ROW):
            col0 = half * CHUNK
            for v in range(NVEC):
                xc = lin_v[pl.ds(col0 + v * L, L)]
                sxn = t00 * xc + t01 * yc + t02
                syn = t10 * xc + t11 * yc + t12
                sx = 0.5 * (sxn + 1.0) * W
                sy = 0.5 * (syn + 1.0) * H
                x0 = sx.astype(jnp.int32)
                x1 = x0 + 1
                y0 = sy.astype(jnp.int32)
                y1 = y0 + 1
                x0 = jnp.clip(x0, 0, W - 1)
                x1 = jnp.clip(x1, 0, W - 1)
                y0 = jnp.clip(y0, 0, H - 1)
                y1 = jnp.clip(y1, 0, H - 1)
                x0f = x0.astype(jnp.float32)
                x1f = x1.astype(jnp.float32)
                y0f = y0.astype(jnp.float32)
                y1f = y1.astype(jnp.float32)
                sl = pl.ds(v * L, L)
                ia_r[sl] = base_idx + y0 * W + x0
                ib_r[sl] = base_idx + y1 * W + x0
                ic_r[sl] = base_idx + y0 * W + x1
                id_r[sl] = base_idx + y1 * W + x1
                w_r[0, sl] = (x1f - sx) * (y1f - sy)
                w_r[1, sl] = (x1f - sx) * (sy - y0f)
                w_r[2, sl] = (sx - x0f) * (y1f - sy)
                w_r[3, sl] = (sx - x0f) * (sy - y0f)

            cps = [pltpu.async_copy(x_hbm.at[idx], dst, sem)
                   for idx, dst in ((ia_r, ga_r), (ib_r, gb_r),
                                    (ic_r, gc_r), (id_r, gd_r))]
            for cp in cps:
                cp.wait()

            def grp(g, c2):
                sw = pl.ds(g * L, L)
                wa_v = w_r[0, sw]
                wb_v = w_r[1, sw]
                wc_v = w_r[2, sw]
                wd_v = w_r[3, sw]
                for lane in range(L):
                    p = g * L + lane
                    wa = _lane_splat(wa_v, lane)
                    wb = _lane_splat(wb_v, lane)
                    wc = _lane_splat(wc_v, lane)
                    wd = _lane_splat(wd_v, lane)
                    for j in range(C // L):
                        slj = pl.ds(j * L, L)
                        acc = wa * ga_r[p, slj]
                        acc = acc + wb * gb_r[p, slj]
                        acc = acc + wc * gc_r[p, slj]
                        acc = acc + wd * gd_r[p, slj]
                        o_r[p, slj] = acc
                return c2

            lax.fori_loop(0, NVEC, grp, 0)
            pix0 = base_idx + row * OUT_W + col0
            pltpu.sync_copy(o_r, out_hbm.at[pl.ds(pix0, CHUNK)])
        return carry

    lax.fori_loop(0, ROWS_PER_TILE, row_body, 0)


@jax.jit
def _run(x2d, th_pad, lin_bf):
    mesh = plsc.VectorSubcoreMesh(core_axis_name="c", subcore_axis_name="s")
    f = pl.kernel(
        _body,
        out_type=jax.ShapeDtypeStruct((B * HW, C), jnp.float32),
        mesh=mesh,
        scratch_types=[
            pltpu.VMEM((CHUNK,), jnp.int32),
            pltpu.VMEM((CHUNK,), jnp.int32),
            pltpu.VMEM((CHUNK,), jnp.int32),
            pltpu.VMEM((CHUNK,), jnp.int32),
            pltpu.VMEM((4, CHUNK), jnp.float32),
            pltpu.VMEM((CHUNK, C), jnp.float32),
            pltpu.VMEM((CHUNK, C), jnp.float32),
            pltpu.VMEM((CHUNK, C), jnp.float32),
            pltpu.VMEM((CHUNK, C), jnp.float32),
            pltpu.VMEM((CHUNK, C), jnp.float32),
            pltpu.VMEM((L,), jnp.float32),
            pltpu.VMEM((OUT_W + L,), jnp.float32),
            pltpu.SemaphoreType.DMA,
        ],
        compiler_params=pltpu.CompilerParams(use_tc_tiling_on_sc=False),
    )
    return f(x2d, th_pad, lin_bf)


def _round_bf16(a):
    u = lax.bitcast_convert_type(a, jnp.uint32)
    r = (u + jnp.uint32(0x7FFF) + ((u >> 16) & jnp.uint32(1))) & jnp.uint32(0xFFFF0000)
    return lax.bitcast_convert_type(r, jnp.float32)


def kernel(x, transformation):
    x2d = x.reshape(B * HW, C)
    th_bf = _round_bf16(transformation)
    th_pad = jnp.zeros((B, L), jnp.float32).at[:, :6].set(th_bf)
    lin = jnp.linspace(-1.0, 1.0, OUT_W)
    lin_bf = _round_bf16(lin)
    lin_bf = jnp.concatenate([lin_bf, jnp.zeros((L,), jnp.float32)])
    out = _run(x2d, th_pad, lin_bf)
    return out.reshape(B, OUT_H, OUT_W, C)


# speedup vs baseline: 1.3958x; 1.0466x over previous
import jax
import jax.numpy as jnp
from jax import lax
from jax.experimental import pallas as pl
from jax.experimental.pallas import tpu as pltpu
from jax.experimental.pallas import tpu_sc as plsc

B, H, W, C = 8, 224, 224, 96
OUT_H, OUT_W = 224, 224
HW = H * W
NC, NS, L = 2, 16, 16
NW = NC * NS
PIX_PER_TILE = B * OUT_H * OUT_W // NW
ROWS_PER_TILE = PIX_PER_TILE // OUT_W
TILES_PER_BATCH = OUT_H // ROWS_PER_TILE
CHUNK = 112
NVEC = CHUNK // L
CHUNKS_PER_ROW = OUT_W // CHUNK


_SPLAT_DNUMS = lax.GatherDimensionNumbers(
    offset_dims=(), collapsed_slice_dims=(0,), start_index_map=(0,))


def _lane_splat(vec, l):
    idx = jnp.full((L, 1), l, jnp.int32)
    return lax.gather(vec, idx, _SPLAT_DNUMS, slice_sizes=(1,),
                      mode=lax.GatherScatterMode.PROMISE_IN_BOUNDS)


def _lane_splat_dyn(vec, lane):
    idx = (lax.iota(jnp.int32, L) * 0 + lane).reshape(L, 1)
    return lax.gather(vec, idx, _SPLAT_DNUMS, slice_sizes=(1,),
                      mode=lax.GatherScatterMode.PROMISE_IN_BOUNDS)


def _body(x_hbm, th_hbm, lin_hbm, out_hbm,
          ia0, ib0, ic0, id0, ia1, ib1, ic1, id1,
          w0, w1,
          ga0, gb0, gc0, gd0, ga1, gb1, gc1, gd1,
          o0, o1,
          th_v, lin_v, sg0, sg1, so0, so1):
    cid = lax.axis_index("c")
    sid = lax.axis_index("s")
    wid = sid * NC + cid
    b = wid // TILES_PER_BATCH
    q = wid % TILES_PER_BATCH

    pltpu.sync_copy(th_hbm.at[b], th_v)
    pltpu.sync_copy(lin_hbm, lin_v)
    thvec = th_v[...]
    t00 = _lane_splat(thvec, 0)
    t01 = _lane_splat(thvec, 1)
    t02 = _lane_splat(thvec, 2)
    t10 = _lane_splat(thvec, 3)
    t11 = _lane_splat(thvec, 4)
    t12 = _lane_splat(thvec, 5)

    base_idx = b * HW
    idxs = ((ia0, ib0, ic0, id0), (ia1, ib1, ic1, id1))
    gats = ((ga0, gb0, gc0, gd0), (ga1, gb1, gc1, gd1))
    ws = (w0, w1)
    outs = (o0, o1)
    sgs = (sg0, sg1)
    sos = (so0, so1)

    def compute_idx(row, col0, p):
        ia_r, ib_r, ic_r, id_r = idxs[p]
        w_r = ws[p]
        rbase = (row // L) * L
        yc = _lane_splat_dyn(lin_v[pl.ds(rbase, L)], row - rbase)
        for v in range(NVEC):
            xc = lin_v[pl.ds(col0 + v * L, L)]
            sxn = t00 * xc + t01 * yc + t02
            syn = t10 * xc + t11 * yc + t12
            sx = 0.5 * (sxn + 1.0) * W
            sy = 0.5 * (syn + 1.0) * H
            x0 = sx.astype(jnp.int32)
            x1 = x0 + 1
            y0 = sy.astype(jnp.int32)
            y1 = y0 + 1
            x0 = jnp.clip(x0, 0, W - 1)
            x1 = jnp.clip(x1, 0, W - 1)
            y0 = jnp.clip(y0, 0, H - 1)
            y1 = jnp.clip(y1, 0, H - 1)
            x0f = x0.astype(jnp.float32)
            x1f = x1.astype(jnp.float32)
            y0f = y0.astype(jnp.float32)
            y1f = y1.astype(jnp.float32)
            sl = pl.ds(v * L, L)
            ia_r[sl] = base_idx + y0 * W + x0
            ib_r[sl] = base_idx + y1 * W + x0
            ic_r[sl] = base_idx + y0 * W + x1
            id_r[sl] = base_idx + y1 * W + x1
            w_r[0, sl] = (x1f - sx) * (y1f - sy)
            w_r[1, sl] = (x1f - sx) * (sy - y0f)
            w_r[2, sl] = (sx - x0f) * (y1f - sy)
            w_r[3, sl] = (sx - x0f) * (sy - y0f)

    def fire(p):
        for idx, dst in zip(idxs[p], gats[p]):
            pltpu.async_copy(x_hbm.at[idx], dst, sgs[p])

    def drain_gathers(p):
        for idx, dst in zip(idxs[p], gats[p]):
            pltpu.make_async_copy(x_hbm.at[idx], dst, sgs[p]).wait()

    def combine(p):
        ga_r, gb_r, gc_r, gd_r = gats[p]
        w_r = ws[p]
        o_r = outs[p]

        def grp(g, c2):
            sw = pl.ds(g * L, L)
            wa_v = w_r[0, sw]
            wb_v = w_r[1, sw]
            wc_v = w_r[2, sw]
            wd_v = w_r[3, sw]
            for lane in range(L):
                pp = g * L + lane
                wa = _lane_splat(wa_v, lane)
                wb = _lane_splat(wb_v, lane)
                wc = _lane_splat(wc_v, lane)
                wd = _lane_splat(wd_v, lane)
                for j in range(C // L):
                    slj = pl.ds(j * L, L)
                    acc = wa * ga_r[pp, slj]
                    acc = acc + wb * gb_r[pp, slj]
                    acc = acc + wc * gc_r[pp, slj]
                    acc = acc + wd * gd_r[pp, slj]
                    o_r[pp, slj] = acc
            return c2

        lax.fori_loop(0, NVEC, grp, 0)

    row0 = q * ROWS_PER_TILE
    compute_idx(row0, 0, 0)
    fire(0)

    def row_body(r, carry):
        row = q * ROWS_PER_TILE + r
        for h in range(CHUNKS_PER_ROW):
            col0 = h * CHUNK
            nrow = row + h
            ncol = (1 - h) * CHUNK
            if h == 0:
                compute_idx(nrow, ncol, 1 - h)
                fire(1 - h)
            else:
                @pl.when(r < ROWS_PER_TILE - 1)
                def _():
                    compute_idx(nrow, ncol, 1 - h)
                    fire(1 - h)
            drain_gathers(h)

            @pl.when(r > 0)
            def _():
                pltpu.make_async_copy(
                    outs[h], out_hbm.at[pl.ds(0, CHUNK)], sos[h]).wait()

            combine(h)
            pix0 = base_idx + row * OUT_W + col0
            pltpu.async_copy(outs[h], out_hbm.at[pl.ds(pix0, CHUNK)], sos[h])
        return carry

    lax.fori_loop(0, ROWS_PER_TILE, row_body, 0)
    for h in range(2):
        pltpu.make_async_copy(outs[h], out_hbm.at[pl.ds(0, CHUNK)],
                              sos[h]).wait()


@jax.jit
def _run(x2d, th_pad, lin_bf):
    mesh = plsc.VectorSubcoreMesh(core_axis_name="c", subcore_axis_name="s")
    f = pl.kernel(
        _body,
        out_type=jax.ShapeDtypeStruct((B * HW, C), jnp.float32),
        mesh=mesh,
        scratch_types=(
            [pltpu.VMEM((CHUNK,), jnp.int32)] * 8
            + [pltpu.VMEM((4, CHUNK), jnp.float32)] * 2
            + [pltpu.VMEM((CHUNK, C), jnp.float32)] * 8
            + [pltpu.VMEM((CHUNK, C), jnp.float32)] * 2
            + [pltpu.VMEM((L,), jnp.float32),
               pltpu.VMEM((OUT_W + L,), jnp.float32),
               pltpu.SemaphoreType.DMA,
               pltpu.SemaphoreType.DMA,
               pltpu.SemaphoreType.DMA,
               pltpu.SemaphoreType.DMA]
        ),
        compiler_params=pltpu.CompilerParams(use_tc_tiling_on_sc=False),
    )
    return f(x2d, th_pad, lin_bf)


def _round_bf16(a):
    u = lax.bitcast_convert_type(a, jnp.uint32)
    r = (u + jnp.uint32(0x7FFF) + ((u >> 16) & jnp.uint32(1))) & jnp.uint32(0xFFFF0000)
    return lax.bitcast_convert_type(r, jnp.float32)


def kernel(x, transformation):
    x2d = x.reshape(B * HW, C)
    th_bf = _round_bf16(transformation)
    th_pad = jnp.zeros((B, L), jnp.float32).at[:, :6].set(th_bf)
    lin = jnp.linspace(-1.0, 1.0, OUT_W)
    lin_bf = _round_bf16(lin)
    lin_bf = jnp.concatenate([lin_bf, jnp.zeros((L,), jnp.float32)])
    out = _run(x2d, th_pad, lin_bf)
    return out.reshape(B, OUT_H, OUT_W, C)


# speedup vs baseline: 1.4167x; 1.0150x over previous
import jax
import jax.numpy as jnp
from jax import lax
from jax.experimental import pallas as pl
from jax.experimental.pallas import tpu as pltpu
from jax.experimental.pallas import tpu_sc as plsc

B, H, W, C = 8, 224, 224, 96
OUT_H, OUT_W = 224, 224
HW = H * W
NC, NS, L = 2, 16, 16
NW = NC * NS
PIX_PER_TILE = B * OUT_H * OUT_W // NW
ROWS_PER_TILE = PIX_PER_TILE // OUT_W
TILES_PER_BATCH = OUT_H // ROWS_PER_TILE
CHUNK = 112
NVEC = CHUNK // L
CHUNKS_PER_ROW = OUT_W // CHUNK


_SPLAT_DNUMS = lax.GatherDimensionNumbers(
    offset_dims=(), collapsed_slice_dims=(0,), start_index_map=(0,))


def _lane_splat(vec, l):
    idx = jnp.full((L, 1), l, jnp.int32)
    return lax.gather(vec, idx, _SPLAT_DNUMS, slice_sizes=(1,),
                      mode=lax.GatherScatterMode.PROMISE_IN_BOUNDS)


def _lane_splat_dyn(vec, lane):
    idx = (lax.iota(jnp.int32, L) * 0 + lane).reshape(L, 1)
    return lax.gather(vec, idx, _SPLAT_DNUMS, slice_sizes=(1,),
                      mode=lax.GatherScatterMode.PROMISE_IN_BOUNDS)


def _body(x_hbm, th_hbm, lin_hbm, out_hbm,
          ia0, ib0, ic0, id0, ia1, ib1, ic1, id1,
          w0, w1,
          ga0, gb0, gc0, gd0, ga1, gb1, gc1, gd1,
          o0, o1,
          th_v, lin_v, sg0, sg1, so0, so1):
    cid = lax.axis_index("c")
    sid = lax.axis_index("s")
    wid = sid * NC + cid
    b = wid // TILES_PER_BATCH
    q = wid % TILES_PER_BATCH

    pltpu.sync_copy(th_hbm.at[b], th_v)
    pltpu.sync_copy(lin_hbm, lin_v)
    thvec = th_v[...]
    t00 = _lane_splat(thvec, 0)
    t01 = _lane_splat(thvec, 1)
    t02 = _lane_splat(thvec, 2)
    t10 = _lane_splat(thvec, 3)
    t11 = _lane_splat(thvec, 4)
    t12 = _lane_splat(thvec, 5)

    base_idx = b * HW
    idxs = ((ia0, ib0, ic0, id0), (ia1, ib1, ic1, id1))
    gats = ((ga0, gb0, gc0, gd0), (ga1, gb1, gc1, gd1))
    ws = (w0, w1)
    outs = (o0, o1)
    sgs = (sg0, sg1)
    sos = (so0, so1)

    def compute_idx(row, col0, p):
        ia_r, ib_r, ic_r, id_r = idxs[p]
        w_r = ws[p]
        rbase = (row // L) * L
        yc = _lane_splat_dyn(lin_v[pl.ds(rbase, L)], row - rbase)
        for v in range(NVEC):
            xc = lin_v[pl.ds(col0 + v * L, L)]
            sxn = t00 * xc + t01 * yc + t02
            syn = t10 * xc + t11 * yc + t12
            sx = 0.5 * (sxn + 1.0) * W
            sy = 0.5 * (syn + 1.0) * H
            x0 = sx.astype(jnp.int32)
            x1 = x0 + 1
            y0 = sy.astype(jnp.int32)
            y1 = y0 + 1
            x0 = jnp.clip(x0, 0, W - 1)
            x1 = jnp.clip(x1, 0, W - 1)
            y0 = jnp.clip(y0, 0, H - 1)
            y1 = jnp.clip(y1, 0, H - 1)
            x0f = x0.astype(jnp.float32)
            x1f = x1.astype(jnp.float32)
            y0f = y0.astype(jnp.float32)
            y1f = y1.astype(jnp.float32)
            sl = pl.ds(v * L, L)
            ia_r[sl] = base_idx + y0 * W + x0
            ib_r[sl] = base_idx + y1 * W + x0
            ic_r[sl] = base_idx + y0 * W + x1
            id_r[sl] = base_idx + y1 * W + x1
            w_r[0, sl] = (x1f - sx) * (y1f - sy)
            w_r[1, sl] = (x1f - sx) * (sy - y0f)
            w_r[2, sl] = (sx - x0f) * (y1f - sy)
            w_r[3, sl] = (sx - x0f) * (sy - y0f)

    def fire(p):
        for idx, dst in zip(idxs[p], gats[p]):
            pltpu.async_copy(x_hbm.at[idx], dst, sgs[p])

    def drain_gathers(p):
        for idx, dst in zip(idxs[p], gats[p]):
            pltpu.make_async_copy(x_hbm.at[idx], dst, sgs[p]).wait()

    def combine(p):
        ga_r, gb_r, gc_r, gd_r = gats[p]
        w_r = ws[p]
        o_r = outs[p]

        @plsc.parallel_loop(0, CHUNK, 1, unroll=4)
        def px(pp):
            v16 = (pp // L) * L
            lane = pp - v16
            sw = pl.ds(v16, L)
            idx = (lax.iota(jnp.int32, L) * 0 + lane).reshape(L, 1)
            gat = lambda vec: lax.gather(
                vec, idx, _SPLAT_DNUMS, slice_sizes=(1,),
                mode=lax.GatherScatterMode.PROMISE_IN_BOUNDS)
            wa = gat(w_r[0, sw])
            wb = gat(w_r[1, sw])
            wc = gat(w_r[2, sw])
            wd = gat(w_r[3, sw])
            for j in range(C // L):
                slj = pl.ds(j * L, L)
                acc = wa * ga_r[pp, slj]
                acc = acc + wb * gb_r[pp, slj]
                acc = acc + wc * gc_r[pp, slj]
                acc = acc + wd * gd_r[pp, slj]
                o_r[pp, slj] = acc

    row0 = q * ROWS_PER_TILE
    compute_idx(row0, 0, 0)
    fire(0)

    def row_body(r, carry):
        row = q * ROWS_PER_TILE + r
        for h in range(CHUNKS_PER_ROW):
            col0 = h * CHUNK
            nrow = row + h
            ncol = (1 - h) * CHUNK
            if h == 0:
                compute_idx(nrow, ncol, 1 - h)
                fire(1 - h)
            else:
                @pl.when(r < ROWS_PER_TILE - 1)
                def _():
                    compute_idx(nrow, ncol, 1 - h)
                    fire(1 - h)
            drain_gathers(h)

            @pl.when(r > 0)
            def _():
                pltpu.make_async_copy(
                    outs[h], out_hbm.at[pl.ds(0, CHUNK)], sos[h]).wait()

            combine(h)
            pix0 = base_idx + row * OUT_W + col0
            pltpu.async_copy(outs[h], out_hbm.at[pl.ds(pix0, CHUNK)], sos[h])
        return carry

    lax.fori_loop(0, ROWS_PER_TILE, row_body, 0)
    for h in range(2):
        pltpu.make_async_copy(outs[h], out_hbm.at[pl.ds(0, CHUNK)],
                              sos[h]).wait()


@jax.jit
def _run(x2d, th_pad, lin_bf):
    mesh = plsc.VectorSubcoreMesh(core_axis_name="c", subcore_axis_name="s")
    f = pl.kernel(
        _body,
        out_type=jax.ShapeDtypeStruct((B * HW, C), jnp.float32),
        mesh=mesh,
        scratch_types=(
            [pltpu.VMEM((CHUNK,), jnp.int32)] * 8
            + [pltpu.VMEM((4, CHUNK), jnp.float32)] * 2
            + [pltpu.VMEM((CHUNK, C), jnp.float32)] * 8
            + [pltpu.VMEM((CHUNK, C), jnp.float32)] * 2
            + [pltpu.VMEM((L,), jnp.float32),
               pltpu.VMEM((OUT_W + L,), jnp.float32),
               pltpu.SemaphoreType.DMA,
               pltpu.SemaphoreType.DMA,
               pltpu.SemaphoreType.DMA,
               pltpu.SemaphoreType.DMA]
        ),
        compiler_params=pltpu.CompilerParams(use_tc_tiling_on_sc=False),
    )
    return f(x2d, th_pad, lin_bf)


def _round_bf16(a):
    u = lax.bitcast_convert_type(a, jnp.uint32)
    r = (u + jnp.uint32(0x7FFF) + ((u >> 16) & jnp.uint32(1))) & jnp.uint32(0xFFFF0000)
    return lax.bitcast_convert_type(r, jnp.float32)


def kernel(x, transformation):
    x2d = x.reshape(B * HW, C)
    th_bf = _round_bf16(transformation)
    th_pad = jnp.zeros((B, L), jnp.float32).at[:, :6].set(th_bf)
    lin = jnp.linspace(-1.0, 1.0, OUT_W)
    lin_bf = _round_bf16(lin)
    lin_bf = jnp.concatenate([lin_bf, jnp.zeros((L,), jnp.float32)])
    out = _run(x2d, th_pad, lin_bf)
    return out.reshape(B, OUT_H, OUT_W, C)
